# Initial kernel scaffold; baseline (speedup 1.0000x reference)
import jax
import jax.numpy as jnp
from jax.experimental import pallas as pl


def kernel(x, loc, ln_w, ln_b, W_conf, b_conf, W_pos, b_pos, H, W, N_grid):
    raise NotImplementedError("write your pallas kernel here")


# speedup vs baseline: 6.3383x; 6.3383x over previous
import jax
import jax.numpy as jnp
from jax.experimental import pallas as pl
from jax.experimental.pallas import tpu as pltpu

_B, _N, _C = 8, 4096, 128
_NG = 1024
_NA = _N - _NG
_K = 1024
_HS = 64
_WS = 64
_HW = _HS * _WS
_CH = 512


def _select_body(srow_ref, scol_ref, loct_ref, out_ref):
    rank_rows = []
    for it in range(0, _NA, _CH):
        acc = jnp.zeros((1, _CH), jnp.float32)
        srow = srow_ref[0, 0:1, it:it + _CH]
        iio = jax.lax.broadcasted_iota(jnp.int32, (1, _CH), 1) + it
        for jt in range(0, _NA, _CH):
            scol = scol_ref[0, jt:jt + _CH, :]
            jio = jax.lax.broadcasted_iota(jnp.int32, (_CH, 1), 0) + jt
            gt = (scol > srow) | ((scol == srow) & (jio < iio))
            acc = acc + jnp.sum(gt.astype(jnp.float32), axis=0, keepdims=True)
        rank_rows.append(acc)
    rcol = jax.lax.broadcasted_iota(jnp.int32, (_K, 1), 0).astype(jnp.float32)
    accx = jnp.zeros((_K, 1), jnp.float32)
    accy = jnp.zeros((_K, 1), jnp.float32)
    for t, it in enumerate(range(0, _NA, _CH)):
        oh = (rank_rows[t] == rcol).astype(jnp.float32)
        lx = loct_ref[0, 0:1, it:it + _CH]
        ly = loct_ref[0, 1:2, it:it + _CH]
        accx = accx + jnp.sum(oh * lx, axis=1, keepdims=True)
        accy = accy + jnp.sum(oh * ly, axis=1, keepdims=True)
    out_ref[0, :, 0:1] = accx
    out_ref[0, :, 1:2] = accy


def _scatter_body(x_ref, loct_ref, feat_ref, cnt_ref):
    ct = pl.program_id(1)
    lx = jnp.clip(loct_ref[0, 0:1, :], -1.0, 1.0)
    ly = jnp.clip(loct_ref[0, 1:2, :], -1.0, 1.0)
    pxf = 0.5 * (lx + 1.0) * 64.0 - 0.5
    pyf = 0.5 * (ly + 1.0) * 64.0 - 0.5
    xi = jnp.clip(jnp.round(pxf).astype(jnp.int32), 0, _WS - 1)
    yi = jnp.clip(jnp.round(pyf).astype(jnp.int32), 0, _HS - 1)
    cell = xi + yi * _WS
    cell_col = jax.lax.broadcasted_iota(jnp.int32, (_CH, 1), 0) + ct * _CH
    acc = jnp.zeros((_CH, _C), jnp.float32)
    cnt = jnp.zeros((_CH, 1), jnp.float32)
    for nt in range(0, _N, _CH):
        oh = (cell[:, nt:nt + _CH] == cell_col).astype(jnp.float32)
        acc = acc + jnp.dot(oh, x_ref[0, nt:nt + _CH, :],
                            preferred_element_type=jnp.float32)
        cnt = cnt + jnp.sum(oh, axis=1, keepdims=True)
    feat_ref[0] = acc
    cnt_ref[0] = cnt


_GK = None


def _gauss_weights():
    import math as _math
    import numpy as _np
    coords = _np.arange(3, dtype=_np.float32)
    x_grid = _np.tile(coords, 3).reshape(3, 3)
    y_grid = x_grid.T
    mean, variance = 1.0, 4.0
    gk = (1.0 / (2.0 * _math.pi * variance)
          * _np.exp(-((x_grid - mean) ** 2 + (y_grid - mean) ** 2)
                    / (2.0 * variance)))
    gk = gk / gk.sum()
    return gk.astype(_np.float32)


def _blur_body(feat_ref, cnt_ref, out_ref):
    gk = _gauss_weights()
    cnt = cnt_ref[0]
    mask = (cnt > 0).astype(jnp.float32)
    feature = feat_ref[0] / (cnt + 1e-6) * mask
    zf = jnp.zeros((65, _C), jnp.float32)
    zm = jnp.zeros((65, 1), jnp.float32)
    fp = jnp.concatenate([zf, feature, zf], axis=0)
    mp = jnp.concatenate([zm, mask, zm], axis=0)
    xpos = jax.lax.broadcasted_iota(jnp.int32, (_HW, 1), 0) & (_WS - 1)
    accf = jnp.zeros((_HW, _C), jnp.float32)
    accm = jnp.zeros((_HW, 1), jnp.float32)
    for dy in (-1, 0, 1):
        for dx in (-1, 0, 1):
            w = float(gk[dy + 1, dx + 1])
            o = 65 + dy * _WS + dx
            if dx == -1:
                xm = (xpos >= 1).astype(jnp.float32)
            elif dx == 1:
                xm = (xpos <= _WS - 2).astype(jnp.float32)
            else:
                xm = None
            fs = fp[o:o + _HW, :]
            ms = mp[o:o + _HW, :]
            if xm is not None:
                fs = fs * xm
                ms = ms * xm
            accf = accf + w * fs
            accm = accm + w * ms
    fi = accf / (accm + 1e-6)
    mi = (accm > 0).astype(jnp.float32)
    fi = fi * mi
    out_ref[0] = feature + (1.0 - mask) * fi


def _gather_body(xmap_ref, ld_ref, xg_ref, lg_ref, pw_ref, pb_ref, out_ref):
    pw0 = pw_ref[0:1, :]
    pw1 = pw_ref[1:2, :]
    pb = pb_ref[...]
    lgx = lg_ref[0, :, 0:1]
    lgy = lg_ref[0, :, 1:2]
    out_ref[0, 0:_NG, :] = xg_ref[0] + (lgx * pw0 + lgy * pw1 + pb)
    lx = ld_ref[0, :, 0:1]
    ly = ld_ref[0, :, 1:2]
    px = (lx + 1.0) * 0.5 * 64.0 - 0.5
    py = (ly + 1.0) * 0.5 * 64.0 - 0.5
    x0 = jnp.floor(px)
    y0 = jnp.floor(py)
    x1 = x0 + 1.0
    y1 = y0 + 1.0
    wx1 = px - x0
    wx0 = 1.0 - wx1
    wy1 = py - y0
    wy0 = 1.0 - wy1
    corners = ((x0, y0, wx0 * wy0), (x1, y0, wx1 * wy0),
               (x0, y1, wx0 * wy1), (x1, y1, wx1 * wy1))
    cws = []
    for xf, yf, w in corners:
        valid = ((xf >= 0) & (xf < _WS) & (yf >= 0) & (yf < _HS))
        xc = jnp.clip(xf, 0, _WS - 1).astype(jnp.int32)
        yc = jnp.clip(yf, 0, _HS - 1).astype(jnp.int32)
        cellc = yc * _WS + xc
        cws.append((cellc, w * valid.astype(jnp.float32)))
    acc = jnp.zeros((_K, _C), jnp.float32)
    for ct in range(0, _HW, _CH):
        ci = jax.lax.broadcasted_iota(jnp.int32, (1, _CH), 1) + ct
        oh = jnp.zeros((_K, _CH), jnp.float32)
        for cellc, w in cws:
            oh = oh + w * (cellc == ci).astype(jnp.float32)
        acc = acc + jnp.dot(oh, xmap_ref[0, ct:ct + _CH, :],
                            preferred_element_type=jnp.float32)
    out_ref[0, _NG:, :] = acc + (lx * pw0 + ly * pw1 + pb)


def kernel(x, loc, ln_w, ln_b, W_conf, b_conf, W_pos, b_pos, H, W, N_grid):
    del H, W, N_grid
    f32 = jnp.float32
    x = x.astype(f32)
    loc = loc.astype(f32)

    u = jax.random.uniform(jax.random.key(42), (_B, _NA), dtype=f32)
    nz = -1.0 * jnp.log(u + 1e-6)
    nz = -1.0 * jnp.log(nz + 1e-6)

    loct = jnp.transpose(loc, (0, 2, 1))
    loct_ada = loct[:, :, _NG:]

    mu = jnp.mean(x, axis=-1, keepdims=True)
    var = jnp.mean((x - mu) ** 2, axis=-1, keepdims=True)
    y = (x - mu) / jnp.sqrt(var + 1e-5) * ln_w + ln_b
    conf = y @ W_conf.T + b_conf
    scores = conf[:, _NG:, 0] + nz

    loc_down = pl.pallas_call(
        _select_body,
        grid=(_B,),
        in_specs=[
            pl.BlockSpec((1, 1, _NA), lambda b: (b, 0, 0)),
            pl.BlockSpec((1, _NA, 1), lambda b: (b, 0, 0)),
            pl.BlockSpec((1, 2, _NA), lambda b: (b, 0, 0)),
        ],
        out_specs=pl.BlockSpec((1, _K, 2), lambda b: (b, 0, 0)),
        out_shape=jax.ShapeDtypeStruct((_B, _K, 2), f32),
    )(scores.reshape(_B, 1, _NA), scores.reshape(_B, _NA, 1), loct_ada)

    feat, cnt = pl.pallas_call(
        _scatter_body,
        grid=(_B, _HW // _CH),
        in_specs=[
            pl.BlockSpec((1, _N, _C), lambda b, ct: (b, 0, 0)),
            pl.BlockSpec((1, 2, _N), lambda b, ct: (b, 0, 0)),
        ],
        out_specs=[
            pl.BlockSpec((1, _CH, _C), lambda b, ct: (b, ct, 0)),
            pl.BlockSpec((1, _CH, 1), lambda b, ct: (b, ct, 0)),
        ],
        out_shape=[
            jax.ShapeDtypeStruct((_B, _HW, _C), f32),
            jax.ShapeDtypeStruct((_B, _HW, 1), f32),
        ],
    )(x, loct)

    xmap = pl.pallas_call(
        _blur_body,
        grid=(_B,),
        in_specs=[
            pl.BlockSpec((1, _HW, _C), lambda b: (b, 0, 0)),
            pl.BlockSpec((1, _HW, 1), lambda b: (b, 0, 0)),
        ],
        out_specs=pl.BlockSpec((1, _HW, _C), lambda b: (b, 0, 0)),
        out_shape=jax.ShapeDtypeStruct((_B, _HW, _C), f32),
    )(feat, cnt)

    out = pl.pallas_call(
        _gather_body,
        grid=(_B,),
        in_specs=[
            pl.BlockSpec((1, _HW, _C), lambda b: (b, 0, 0)),
            pl.BlockSpec((1, _K, 2), lambda b: (b, 0, 0)),
            pl.BlockSpec((1, _NG, _C), lambda b: (b, 0, 0)),
            pl.BlockSpec((1, _NG, 2), lambda b: (b, 0, 0)),
            pl.BlockSpec((2, _C), lambda b: (0, 0)),
            pl.BlockSpec((1, _C), lambda b: (0, 0)),
        ],
        out_specs=pl.BlockSpec((1, 2 * _K, _C), lambda b: (b, 0, 0)),
        out_shape=jax.ShapeDtypeStruct((_B, 2 * _K, _C), f32),
    )(xmap, loc_down, x[:, :_NG], loc[:, :_NG],
      jnp.transpose(W_pos.astype(f32)), b_pos.reshape(1, _C).astype(f32))

    return out
